# Initial kernel scaffold; baseline (speedup 1.0000x reference)
import jax
import jax.numpy as jnp
from jax.experimental import pallas as pl


def kernel(x, edge_index, W_in, b_in, W1, b1, g1, beta1, W2, b2, g2, beta2, W3, b3, g3, beta3, W_out, b_out):
    raise NotImplementedError("write your pallas kernel here")


# speedup vs baseline: 2.9773x; 2.9773x over previous
import functools

import jax
import jax.numpy as jnp
from jax import lax
from jax.experimental import pallas as pl
from jax.experimental.pallas import tpu as pltpu
from jax.experimental.pallas import tpu_sc as plsc

N = 10000
E = 160000
D_IN = 256
D_H = 512
C = 4
DC = D_H // C
NPAD = 10240
EPAD = 163840
TILES = 32
EPT = EPAD // TILES
KB = EPT // 128
R = 1000
NR = N // R
_PREC = lax.Precision.HIGHEST

def _mesh():
    return plsc.VectorSubcoreMesh(core_axis_name="c", subcore_axis_name="s")


def _sc_degree(dsts, ones128, zeros128):
    @functools.partial(
        pl.kernel,
        out_type=jax.ShapeDtypeStruct((2, NPAD, DC), jnp.float32),
        mesh=_mesh(),
        scratch_types=[
            pltpu.VMEM((KB, 128), jnp.int32),
            pltpu.VMEM((128, DC), jnp.float32),
            pltpu.VMEM((128, DC), jnp.float32),
            pltpu.VMEM_SHARED((NPAD, DC), jnp.float32),
        ],
    )
    def k(dsts_hbm, ones_hbm, zeros_hbm, deg_hbm, didx, ones_v, zz, acc):
        kc = lax.axis_index("c")
        s = lax.axis_index("s")
        t = kc * 16 + s
        pltpu.sync_copy(dsts_hbm.at[t], didx)
        pltpu.sync_copy(ones_hbm, ones_v)
        pltpu.sync_copy(zeros_hbm, zz)
        for z in range(5):
            pltpu.sync_copy(zz, acc.at[pl.ds(s * (NPAD // 16) + z * 128, 128), :])
        plsc.subcore_barrier()

        @pl.loop(0, KB)
        def _(j):
            pltpu.sync_copy(ones_v, acc.at[didx.at[j]], add=True)

        plsc.subcore_barrier()
        for z in range(5):
            off = s * (NPAD // 16) + z * 128
            pltpu.sync_copy(acc.at[pl.ds(off, 128), :], ones_v)
            pltpu.sync_copy(ones_v, deg_hbm.at[kc, pl.ds(off, 128), :])

    return k(dsts, ones128, zeros128)


def _sc_scatter(srcs, dsts, g2, zeros128):
    @functools.partial(
        pl.kernel,
        out_type=jax.ShapeDtypeStruct((2, C, NPAD, DC), jnp.float32),
        mesh=_mesh(),
        scratch_types=[
            pltpu.VMEM((KB, 128), jnp.int32),
            pltpu.VMEM((KB, 128), jnp.int32),
            pltpu.VMEM((128, DC), jnp.float32),
            pltpu.VMEM((128, DC), jnp.float32),
            pltpu.VMEM_SHARED((NPAD, DC), jnp.float32),
        ],
    )
    def k(srcs_hbm, dsts_hbm, g2_hbm, zeros_hbm, part_hbm,
          sidx, didx, rows, zz, acc):
        kc = lax.axis_index("c")
        s = lax.axis_index("s")
        t = kc * 16 + s
        pltpu.sync_copy(dsts_hbm.at[t], didx)
        pltpu.sync_copy(srcs_hbm.at[t], sidx)
        pltpu.sync_copy(zeros_hbm, zz)
        for c in range(C):
            for z in range(5):
                pltpu.sync_copy(zz, acc.at[pl.ds(s * (NPAD // 16) + z * 128, 128), :])

            if c > 0:
                @pl.loop(0, KB)
                def _(j):
                    for kk in range(8):
                        sl = pl.ds(kk * 16, 16)
                        sidx[j, sl] = sidx[j, sl] + N

            plsc.subcore_barrier()

            @pl.loop(0, KB)
            def _(j):
                pltpu.sync_copy(g2_hbm.at[sidx.at[j]], rows)
                pltpu.sync_copy(rows, acc.at[didx.at[j]], add=True)

            plsc.subcore_barrier()
            for z in range(5):
                off = s * (NPAD // 16) + z * 128
                pltpu.sync_copy(acc.at[pl.ds(off, 128), :], rows)
                pltpu.sync_copy(rows, part_hbm.at[kc, c, pl.ds(off, 128), :])
            plsc.subcore_barrier()

    return k(srcs, dsts, g2, zeros128)


def _tc_in_proj(x, W_in, bb, deg):
    def body(x_ref, w_ref, b_ref, deg_ref, h_ref, dinv_ref):
        acc = jnp.dot(x_ref[...], w_ref[...], precision=_PREC)
        h_ref[0] = jnp.maximum(acc + b_ref[0, 0:1, :], 0.0)
        d = deg_ref[0, :, 0:1] + deg_ref[1, :, 0:1] + 1.0
        dinv_ref[...] = jnp.broadcast_to(lax.rsqrt(d), (R, DC))

    return pl.pallas_call(
        body,
        grid=(NR, C),
        in_specs=[
            pl.BlockSpec((R, D_IN), lambda i, co: (i, 0)),
            pl.BlockSpec((D_IN, DC), lambda i, co: (0, co)),
            pl.BlockSpec((1, 8, DC), lambda i, co: (co, 0, 0)),
            pl.BlockSpec((2, R, DC), lambda i, co: (0, i, 0)),
        ],
        out_specs=[
            pl.BlockSpec((1, R, DC), lambda i, co: (co, i, 0)),
            pl.BlockSpec((R, DC), lambda i, co: (i, 0)),
        ],
        out_shape=[
            jax.ShapeDtypeStruct((C, N, DC), jnp.float32),
            jax.ShapeDtypeStruct((N, DC), jnp.float32),
        ],
    )(x, W_in, bb, deg)


def _tc_linscale(h, Wr, dinv):
    def body(h_ref, w_ref, dinv_ref, g_ref):
        acc = jnp.dot(h_ref[0], w_ref[0], precision=_PREC)
        for c in range(1, C):
            acc += jnp.dot(h_ref[c], w_ref[c], precision=_PREC)
        g_ref[0] = dinv_ref[...] * acc

    return pl.pallas_call(
        body,
        grid=(NR, C),
        in_specs=[
            pl.BlockSpec((C, R, DC), lambda i, co: (0, i, 0)),
            pl.BlockSpec((C, DC, DC), lambda i, co: (0, 0, co)),
            pl.BlockSpec((R, DC), lambda i, co: (i, 0)),
        ],
        out_specs=pl.BlockSpec((1, R, DC), lambda i, co: (co, i, 0)),
        out_shape=jax.ShapeDtypeStruct((C, N, DC), jnp.float32),
    )(h, Wr, dinv)


def _tc_combine(part, g, dinv, bb):
    def body(p_ref, g_ref, dinv_ref, b_ref, conv_ref, st_ref):
        i = pl.program_id(1)
        conv = dinv_ref[...] * (p_ref[0, 0] + p_ref[1, 0] + g_ref[0]) \
            + b_ref[0, 0:1, :]
        conv_ref[0] = conv
        ssum = jnp.sum(conv.reshape(R // 8, 8, DC), axis=0)
        ssq = jnp.sum((conv * conv).reshape(R // 8, 8, DC), axis=0)

        @pl.when(i == 0)
        def _():
            st_ref[0, 0] = ssum
            st_ref[0, 1] = ssq

        @pl.when(i != 0)
        def _():
            st_ref[0, 0] += ssum
            st_ref[0, 1] += ssq

    return pl.pallas_call(
        body,
        grid=(C, NR),
        in_specs=[
            pl.BlockSpec((2, 1, R, DC), lambda c, i: (0, c, i, 0)),
            pl.BlockSpec((1, R, DC), lambda c, i: (c, i, 0)),
            pl.BlockSpec((R, DC), lambda c, i: (i, 0)),
            pl.BlockSpec((1, 8, DC), lambda c, i: (c, 0, 0)),
        ],
        out_specs=[
            pl.BlockSpec((1, R, DC), lambda c, i: (c, i, 0)),
            pl.BlockSpec((1, 2, 8, DC), lambda c, i: (c, 0, 0, 0)),
        ],
        out_shape=[
            jax.ShapeDtypeStruct((C, N, DC), jnp.float32),
            jax.ShapeDtypeStruct((C, 2, 8, DC), jnp.float32),
        ],
    )(part, g, dinv, bb)


def _tc_bn_relu_res(conv, stats, gam, bet, hprev):
    def body(conv_ref, st_ref, g_ref, b_ref, hp_ref, h_ref):
        mu = jnp.sum(st_ref[0, 0], axis=0, keepdims=True) * (1.0 / N)
        ex2 = jnp.sum(st_ref[0, 1], axis=0, keepdims=True) * (1.0 / N)
        var = ex2 - mu * mu
        inv = lax.rsqrt(var + 1e-5)
        y = (conv_ref[0] - mu) * inv * g_ref[0, 0:1, :] + b_ref[0, 0:1, :]
        h_ref[0] = jnp.maximum(y, 0.0) + hp_ref[0]

    return pl.pallas_call(
        body,
        grid=(C, NR),
        in_specs=[
            pl.BlockSpec((1, R, DC), lambda c, i: (c, i, 0)),
            pl.BlockSpec((1, 2, 8, DC), lambda c, i: (c, 0, 0, 0)),
            pl.BlockSpec((1, 8, DC), lambda c, i: (c, 0, 0)),
            pl.BlockSpec((1, 8, DC), lambda c, i: (c, 0, 0)),
            pl.BlockSpec((1, R, DC), lambda c, i: (c, i, 0)),
        ],
        out_specs=pl.BlockSpec((1, R, DC), lambda c, i: (c, i, 0)),
        out_shape=jax.ShapeDtypeStruct((C, N, DC), jnp.float32),
    )(conv, stats, gam, bet, hprev)


def _tc_out(h, Wo, bo):
    def body(h_ref, w_ref, b_ref, o_ref):
        acc = jnp.dot(h_ref[0], w_ref[0], precision=_PREC)
        for c in range(1, C):
            acc += jnp.dot(h_ref[c], w_ref[c], precision=_PREC)
        o_ref[...] = acc + b_ref[0:1, :]

    return pl.pallas_call(
        body,
        grid=(NR,),
        in_specs=[
            pl.BlockSpec((C, R, DC), lambda i: (0, i, 0)),
            pl.BlockSpec((C, DC, 128), lambda i: (0, 0, 0)),
            pl.BlockSpec((8, 128), lambda i: (0, 0)),
        ],
        out_specs=pl.BlockSpec((R, 128), lambda i: (i, 0)),
        out_shape=jax.ShapeDtypeStruct((N, 128), jnp.float32),
    )(h, Wo, bo)


def _bc(v):
    return jnp.broadcast_to(v.reshape(C, 1, DC), (C, 8, DC))


def kernel(x, edge_index, W_in, b_in, W1, b1, g1, beta1, W2, b2, g2, beta2,
           W3, b3, g3, beta3, W_out, b_out):
    src = edge_index[0]
    dst = edge_index[1]
    pad = EPAD - E
    src_p = jnp.concatenate([src, jnp.zeros((pad,), jnp.int32)])
    dst_p = jnp.concatenate([dst, jnp.full((pad,), N, jnp.int32)])
    dsts = dst_p.reshape(TILES, KB, 128)
    srcs = src_p.reshape(TILES, KB, 128)
    ones128 = jnp.ones((128, DC), jnp.float32)
    zeros128 = jnp.zeros((128, DC), jnp.float32)

    deg = _sc_degree(dsts, ones128, zeros128)
    h, dinv = _tc_in_proj(x, W_in, _bc(b_in), deg)

    for (W, b, gm, be) in ((W1, b1, g1, beta1), (W2, b2, g2, beta2),
                           (W3, b3, g3, beta3)):
        g = _tc_linscale(h, W.reshape(C, DC, D_H), dinv)
        part = _sc_scatter(srcs, dsts, g.reshape(C * N, DC), zeros128)
        conv, stats = _tc_combine(part, g, dinv, _bc(b))
        h = _tc_bn_relu_res(conv, stats, _bc(gm), _bc(be), h)

    Wo = jnp.pad(W_out, ((0, 0), (0, 128 - W_out.shape[1])))
    bo = jnp.broadcast_to(jnp.pad(b_out, (0, 128 - b_out.shape[0]))[None, :],
                          (8, 128))
    out128 = _tc_out(h, Wo.reshape(C, DC, 128), bo)
    return out128[:, :b_out.shape[0]]


# speedup vs baseline: 3.2851x; 1.1034x over previous
import functools

import jax
import jax.numpy as jnp
from jax import lax
from jax.experimental import pallas as pl
from jax.experimental.pallas import tpu as pltpu
from jax.experimental.pallas import tpu_sc as plsc

N = 10000
E = 160000
D_IN = 256
D_H = 512
C = 4
DC = D_H // C
NPAD = 10240
EPAD = 163840
TILES = 32
EPT = EPAD // TILES
KB = EPT // 128
R = 1000
NR = N // R
_PREC = lax.Precision.HIGHEST

def _mesh():
    return plsc.VectorSubcoreMesh(core_axis_name="c", subcore_axis_name="s")


def _sc_degree(dsts, ones128, zeros128):
    @functools.partial(
        pl.kernel,
        out_type=jax.ShapeDtypeStruct((2, NPAD, DC), jnp.float32),
        mesh=_mesh(),
        scratch_types=[
            pltpu.VMEM((KB, 128), jnp.int32),
            pltpu.VMEM((128, DC), jnp.float32),
            pltpu.VMEM((128, DC), jnp.float32),
            pltpu.VMEM_SHARED((NPAD, DC), jnp.float32),
        ],
    )
    def k(dsts_hbm, ones_hbm, zeros_hbm, deg_hbm, didx, ones_v, zz, acc):
        kc = lax.axis_index("c")
        s = lax.axis_index("s")
        t = kc * 16 + s
        pltpu.sync_copy(dsts_hbm.at[t], didx)
        pltpu.sync_copy(ones_hbm, ones_v)
        pltpu.sync_copy(zeros_hbm, zz)
        for z in range(5):
            pltpu.sync_copy(zz, acc.at[pl.ds(s * (NPAD // 16) + z * 128, 128), :])
        plsc.subcore_barrier()

        @pl.loop(0, KB)
        def _(j):
            pltpu.sync_copy(ones_v, acc.at[didx.at[j]], add=True)

        plsc.subcore_barrier()
        for z in range(5):
            off = s * (NPAD // 16) + z * 128
            pltpu.sync_copy(acc.at[pl.ds(off, 128), :], ones_v)
            pltpu.sync_copy(ones_v, deg_hbm.at[kc, pl.ds(off, 128), :])

    return k(dsts, ones128, zeros128)


def _sc_scatter(srcs, dsts, g2, zeros128):
    @functools.partial(
        pl.kernel,
        out_type=jax.ShapeDtypeStruct((2, C, NPAD, DC), jnp.float32),
        mesh=_mesh(),
        scratch_types=[
            pltpu.VMEM((KB, 128), jnp.int32),
            pltpu.VMEM((KB, 128), jnp.int32),
            pltpu.VMEM((128, DC), jnp.float32),
            pltpu.VMEM((128, DC), jnp.float32),
            pltpu.VMEM((32, DC), jnp.float32),
            pltpu.SemaphoreType.DMA,
            pltpu.SemaphoreType.DMA,
            pltpu.VMEM_SHARED((NPAD, DC), jnp.float32),
        ],
    )
    def k(srcs_hbm, dsts_hbm, g2_hbm, zeros_hbm, part_hbm,
          sidx, didx, rows0, rows1, zz, g0, g1, acc):
        kc = lax.axis_index("c")
        s = lax.axis_index("s")
        t = kc * 16 + s
        pltpu.sync_copy(dsts_hbm.at[t], didx)
        pltpu.sync_copy(srcs_hbm.at[t], sidx)
        pltpu.sync_copy(zeros_hbm, zz)
        for c in range(C):
            if c > 0:
                @pl.loop(0, KB)
                def _(j):
                    for kk in range(8):
                        sl = pl.ds(kk * 16, 16)
                        sidx[j, sl] = sidx[j, sl] + N

            for z in range(20):
                pltpu.sync_copy(zz, acc.at[pl.ds(s * (NPAD // 16) + z * 32, 32), :])
            pltpu.async_copy(g2_hbm.at[sidx.at[0]], rows0, g0)
            pltpu.async_copy(g2_hbm.at[sidx.at[1]], rows1, g1)
            plsc.subcore_barrier()

            @pl.loop(0, KB // 2)
            def _(jj):
                for par, rows, sem in ((0, rows0, g0), (1, rows1, g1)):
                    j = jj * 2 + par
                    pltpu.make_async_copy(g2_hbm.at[sidx.at[j]], rows, sem).wait()
                    pltpu.sync_copy(rows, acc.at[didx.at[j]], add=True)

                    @pl.when(j + 2 < KB)
                    def _():
                        pltpu.async_copy(g2_hbm.at[sidx.at[j + 2]], rows, sem)

            plsc.subcore_barrier()
            for z in range(5):
                off = s * (NPAD // 16) + z * 128
                pltpu.sync_copy(acc.at[pl.ds(off, 128), :], rows0)
                pltpu.sync_copy(rows0, part_hbm.at[kc, c, pl.ds(off, 128), :])
            plsc.subcore_barrier()

    return k(srcs, dsts, g2, zeros128)


def _tc_in_proj(x, W_in, bb, deg):
    def body(x_ref, w_ref, b_ref, deg_ref, h_ref, dinv_ref):
        acc = jnp.dot(x_ref[...], w_ref[...], precision=_PREC)
        h_ref[0] = jnp.maximum(acc + b_ref[0, 0:1, :], 0.0)
        d = deg_ref[0, :, 0:1] + deg_ref[1, :, 0:1] + 1.0
        dinv_ref[...] = jnp.broadcast_to(lax.rsqrt(d), (R, DC))

    return pl.pallas_call(
        body,
        grid=(NR, C),
        in_specs=[
            pl.BlockSpec((R, D_IN), lambda i, co: (i, 0)),
            pl.BlockSpec((D_IN, DC), lambda i, co: (0, co)),
            pl.BlockSpec((1, 8, DC), lambda i, co: (co, 0, 0)),
            pl.BlockSpec((2, R, DC), lambda i, co: (0, i, 0)),
        ],
        out_specs=[
            pl.BlockSpec((1, R, DC), lambda i, co: (co, i, 0)),
            pl.BlockSpec((R, DC), lambda i, co: (i, 0)),
        ],
        out_shape=[
            jax.ShapeDtypeStruct((C, N, DC), jnp.float32),
            jax.ShapeDtypeStruct((N, DC), jnp.float32),
        ],
    )(x, W_in, bb, deg)


def _tc_linscale(h, Wr, dinv):
    def body(h_ref, w_ref, dinv_ref, g_ref):
        acc = jnp.dot(h_ref[0], w_ref[0], precision=_PREC)
        for c in range(1, C):
            acc += jnp.dot(h_ref[c], w_ref[c], precision=_PREC)
        g_ref[0] = dinv_ref[...] * acc

    return pl.pallas_call(
        body,
        grid=(NR, C),
        in_specs=[
            pl.BlockSpec((C, R, DC), lambda i, co: (0, i, 0)),
            pl.BlockSpec((C, DC, DC), lambda i, co: (0, 0, co)),
            pl.BlockSpec((R, DC), lambda i, co: (i, 0)),
        ],
        out_specs=pl.BlockSpec((1, R, DC), lambda i, co: (co, i, 0)),
        out_shape=jax.ShapeDtypeStruct((C, N, DC), jnp.float32),
    )(h, Wr, dinv)


def _tc_combine(part, g, dinv, bb):
    def body(p_ref, g_ref, dinv_ref, b_ref, conv_ref, st_ref):
        i = pl.program_id(1)
        conv = dinv_ref[...] * (p_ref[0, 0] + p_ref[1, 0] + g_ref[0]) \
            + b_ref[0, 0:1, :]
        conv_ref[0] = conv
        ssum = jnp.sum(conv.reshape(R // 8, 8, DC), axis=0)
        ssq = jnp.sum((conv * conv).reshape(R // 8, 8, DC), axis=0)

        @pl.when(i == 0)
        def _():
            st_ref[0, 0] = ssum
            st_ref[0, 1] = ssq

        @pl.when(i != 0)
        def _():
            st_ref[0, 0] += ssum
            st_ref[0, 1] += ssq

    return pl.pallas_call(
        body,
        grid=(C, NR),
        in_specs=[
            pl.BlockSpec((2, 1, R, DC), lambda c, i: (0, c, i, 0)),
            pl.BlockSpec((1, R, DC), lambda c, i: (c, i, 0)),
            pl.BlockSpec((R, DC), lambda c, i: (i, 0)),
            pl.BlockSpec((1, 8, DC), lambda c, i: (c, 0, 0)),
        ],
        out_specs=[
            pl.BlockSpec((1, R, DC), lambda c, i: (c, i, 0)),
            pl.BlockSpec((1, 2, 8, DC), lambda c, i: (c, 0, 0, 0)),
        ],
        out_shape=[
            jax.ShapeDtypeStruct((C, N, DC), jnp.float32),
            jax.ShapeDtypeStruct((C, 2, 8, DC), jnp.float32),
        ],
    )(part, g, dinv, bb)


def _tc_bn_relu_res(conv, stats, gam, bet, hprev):
    def body(conv_ref, st_ref, g_ref, b_ref, hp_ref, h_ref):
        mu = jnp.sum(st_ref[0, 0], axis=0, keepdims=True) * (1.0 / N)
        ex2 = jnp.sum(st_ref[0, 1], axis=0, keepdims=True) * (1.0 / N)
        var = ex2 - mu * mu
        inv = lax.rsqrt(var + 1e-5)
        y = (conv_ref[0] - mu) * inv * g_ref[0, 0:1, :] + b_ref[0, 0:1, :]
        h_ref[0] = jnp.maximum(y, 0.0) + hp_ref[0]

    return pl.pallas_call(
        body,
        grid=(C, NR),
        in_specs=[
            pl.BlockSpec((1, R, DC), lambda c, i: (c, i, 0)),
            pl.BlockSpec((1, 2, 8, DC), lambda c, i: (c, 0, 0, 0)),
            pl.BlockSpec((1, 8, DC), lambda c, i: (c, 0, 0)),
            pl.BlockSpec((1, 8, DC), lambda c, i: (c, 0, 0)),
            pl.BlockSpec((1, R, DC), lambda c, i: (c, i, 0)),
        ],
        out_specs=pl.BlockSpec((1, R, DC), lambda c, i: (c, i, 0)),
        out_shape=jax.ShapeDtypeStruct((C, N, DC), jnp.float32),
    )(conv, stats, gam, bet, hprev)


def _tc_out(h, Wo, bo):
    def body(h_ref, w_ref, b_ref, o_ref):
        acc = jnp.dot(h_ref[0], w_ref[0], precision=_PREC)
        for c in range(1, C):
            acc += jnp.dot(h_ref[c], w_ref[c], precision=_PREC)
        o_ref[...] = acc + b_ref[0:1, :]

    return pl.pallas_call(
        body,
        grid=(NR,),
        in_specs=[
            pl.BlockSpec((C, R, DC), lambda i: (0, i, 0)),
            pl.BlockSpec((C, DC, 128), lambda i: (0, 0, 0)),
            pl.BlockSpec((8, 128), lambda i: (0, 0)),
        ],
        out_specs=pl.BlockSpec((R, 128), lambda i: (i, 0)),
        out_shape=jax.ShapeDtypeStruct((N, 128), jnp.float32),
    )(h, Wo, bo)


def _bc(v):
    return jnp.broadcast_to(v.reshape(C, 1, DC), (C, 8, DC))


def kernel(x, edge_index, W_in, b_in, W1, b1, g1, beta1, W2, b2, g2, beta2,
           W3, b3, g3, beta3, W_out, b_out):
    src = edge_index[0]
    dst = edge_index[1]
    pad = EPAD - E
    src_p = jnp.concatenate([src, jnp.zeros((pad,), jnp.int32)])
    dst_p = jnp.concatenate([dst, jnp.full((pad,), N, jnp.int32)])
    dsts = dst_p.reshape(TILES, KB, 128)
    srcs = src_p.reshape(TILES, KB, 128)
    ones128 = jnp.ones((128, DC), jnp.float32)
    zeros128 = jnp.zeros((128, DC), jnp.float32)
    zeros32 = jnp.zeros((32, DC), jnp.float32)

    deg = _sc_degree(dsts, ones128, zeros128)
    h, dinv = _tc_in_proj(x, W_in, _bc(b_in), deg)

    for (W, b, gm, be) in ((W1, b1, g1, beta1), (W2, b2, g2, beta2),
                           (W3, b3, g3, beta3)):
        g = _tc_linscale(h, W.reshape(C, DC, D_H), dinv)
        part = _sc_scatter(srcs, dsts, g.reshape(C * N, DC), zeros32)
        conv, stats = _tc_combine(part, g, dinv, _bc(b))
        h = _tc_bn_relu_res(conv, stats, _bc(gm), _bc(be), h)

    Wo = jnp.pad(W_out, ((0, 0), (0, 128 - W_out.shape[1])))
    bo = jnp.broadcast_to(jnp.pad(b_out, (0, 128 - b_out.shape[0]))[None, :],
                          (8, 128))
    out128 = _tc_out(h, Wo.reshape(C, DC, 128), bo)
    return out128[:, :b_out.shape[0]]


# speedup vs baseline: 3.4607x; 1.0535x over previous
import functools

import jax
import jax.numpy as jnp
from jax import lax
from jax.experimental import pallas as pl
from jax.experimental.pallas import tpu as pltpu
from jax.experimental.pallas import tpu_sc as plsc

N = 10000
E = 160000
D_IN = 256
D_H = 512
C = 4
DC = D_H // C
NPAD = 10112
RPT = NPAD // 16
EPAD = 163840
TILES = 32
EPT = EPAD // TILES
KB = EPT // 128
KB0 = 64
KB1 = 16
R = 1000
NR = N // R
_PREC = lax.Precision.HIGHEST

def _mesh():
    return plsc.VectorSubcoreMesh(core_axis_name="c", subcore_axis_name="s")


def _sc_degree(dsts, ones128, zeros128):
    @functools.partial(
        pl.kernel,
        out_type=jax.ShapeDtypeStruct((2, NPAD, DC), jnp.float32),
        mesh=_mesh(),
        scratch_types=[
            pltpu.VMEM((KB, 128), jnp.int32),
            pltpu.VMEM((128, DC), jnp.float32),
            pltpu.VMEM((128, DC), jnp.float32),
            pltpu.VMEM_SHARED((NPAD, DC), jnp.float32),
        ],
    )
    def k(dsts_hbm, ones_hbm, zeros_hbm, deg_hbm, didx, ones_v, zz, acc):
        kc = lax.axis_index("c")
        s = lax.axis_index("s")
        t = kc * 16 + s
        pltpu.sync_copy(dsts_hbm.at[t], didx)
        pltpu.sync_copy(ones_hbm, ones_v)
        pltpu.sync_copy(zeros_hbm, zz)
        base = s * RPT
        for z in range(4):
            pltpu.sync_copy(zz, acc.at[pl.ds(base + z * 128, 128), :])
        pltpu.sync_copy(zz.at[pl.ds(0, RPT - 512)],
                        acc.at[pl.ds(base + 512, RPT - 512), :])
        plsc.subcore_barrier()

        @pl.loop(0, KB)
        def _(j):
            pltpu.sync_copy(ones_v, acc.at[didx.at[j]], add=True)

        plsc.subcore_barrier()
        for z in range(4):
            off = base + z * 128
            pltpu.sync_copy(acc.at[pl.ds(off, 128), :], ones_v)
            pltpu.sync_copy(ones_v, deg_hbm.at[kc, pl.ds(off, 128), :])
        off = base + 512
        pltpu.sync_copy(acc.at[pl.ds(off, RPT - 512), :],
                        ones_v.at[pl.ds(0, RPT - 512)])
        pltpu.sync_copy(ones_v.at[pl.ds(0, RPT - 512)],
                        deg_hbm.at[kc, pl.ds(off, RPT - 512), :])

    return k(dsts, ones128, zeros128)


def _sc_scatter(srcs, dsts, g2):
    @functools.partial(
        pl.kernel,
        out_type=jax.ShapeDtypeStruct((2, C, NPAD, DC), jnp.float32),
        mesh=_mesh(),
        scratch_types=[
            pltpu.VMEM((KB0, 128), jnp.int32),
            pltpu.VMEM((KB0, 128), jnp.int32),
            pltpu.VMEM((128, DC), jnp.float32),
            pltpu.VMEM((128, DC), jnp.float32),
            pltpu.SemaphoreType.DMA,
            pltpu.SemaphoreType.DMA,
            pltpu.VMEM_SHARED((NPAD, DC), jnp.float32),
        ],
    )
    def k(srcs_hbm, dsts_hbm, g2_hbm, part_hbm,
          sidx, didx, rows0, rows1, g0, g1, acc):
        kc = lax.axis_index("c")
        s = lax.axis_index("s")
        base = s * RPT

        def work(KBc, row_base):
            pltpu.sync_copy(dsts_hbm.at[pl.ds(row_base, KBc)],
                            didx.at[pl.ds(0, KBc)])
            pltpu.sync_copy(srcs_hbm.at[pl.ds(row_base, KBc)],
                            sidx.at[pl.ds(0, KBc)])
            for c in range(C):
                if c > 0:
                    @pl.loop(0, KBc)
                    def _(j):
                        for kk in range(8):
                            sl = pl.ds(kk * 16, 16)
                            sidx[j, sl] = sidx[j, sl] + N

                zv = jnp.zeros((16,), jnp.float32)

                @pl.loop(0, 128)
                def _(i):
                    for kk in range(8):
                        rows1[i, pl.ds(kk * 16, 16)] = zv

                for z in range(4):
                    pltpu.sync_copy(rows1, acc.at[pl.ds(base + z * 128, 128), :])
                pltpu.sync_copy(rows1.at[pl.ds(0, RPT - 512)],
                                acc.at[pl.ds(base + 512, RPT - 512), :])
                pltpu.async_copy(g2_hbm.at[sidx.at[0]], rows0, g0)
                pltpu.async_copy(g2_hbm.at[sidx.at[1]], rows1, g1)
                plsc.subcore_barrier()

                @pl.loop(0, KBc // 2)
                def _(jj):
                    for par, rows, sem in ((0, rows0, g0), (1, rows1, g1)):
                        j = jj * 2 + par
                        pltpu.make_async_copy(
                            g2_hbm.at[sidx.at[j]], rows, sem).wait()
                        pltpu.sync_copy(rows, acc.at[didx.at[j]], add=True)

                        @pl.when(j + 2 < KBc)
                        def _():
                            pltpu.async_copy(g2_hbm.at[sidx.at[j + 2]], rows, sem)

                plsc.subcore_barrier()
                for z in range(4):
                    off = base + z * 128
                    pltpu.sync_copy(acc.at[pl.ds(off, 128), :], rows0)
                    pltpu.sync_copy(rows0, part_hbm.at[kc, c, pl.ds(off, 128), :])
                off = base + 512
                pltpu.sync_copy(acc.at[pl.ds(off, RPT - 512), :],
                                rows0.at[pl.ds(0, RPT - 512)])
                pltpu.sync_copy(rows0.at[pl.ds(0, RPT - 512)],
                                part_hbm.at[kc, c, pl.ds(off, RPT - 512), :])
                plsc.subcore_barrier()

        @pl.when(kc == 0)
        def _():
            work(KB0, s * KB0)

        @pl.when(kc == 1)
        def _():
            work(KB1, 16 * KB0 + s * KB1)

    return k(srcs, dsts, g2)


def _tc_in_proj(x, W_in, bb, deg):
    def body(x_ref, w_ref, b_ref, deg_ref, h_ref, dinv_ref):
        acc = jnp.dot(x_ref[...], w_ref[...], precision=_PREC)
        h_ref[0] = jnp.maximum(acc + b_ref[0, 0:1, :], 0.0)
        d = deg_ref[0, :, 0:1] + deg_ref[1, :, 0:1] + 1.0
        dinv_ref[...] = jnp.broadcast_to(lax.rsqrt(d), (R, DC))

    return pl.pallas_call(
        body,
        grid=(NR, C),
        in_specs=[
            pl.BlockSpec((R, D_IN), lambda i, co: (i, 0)),
            pl.BlockSpec((D_IN, DC), lambda i, co: (0, co)),
            pl.BlockSpec((1, 8, DC), lambda i, co: (co, 0, 0)),
            pl.BlockSpec((2, R, DC), lambda i, co: (0, i, 0)),
        ],
        out_specs=[
            pl.BlockSpec((1, R, DC), lambda i, co: (co, i, 0)),
            pl.BlockSpec((R, DC), lambda i, co: (i, 0)),
        ],
        out_shape=[
            jax.ShapeDtypeStruct((C, N, DC), jnp.float32),
            jax.ShapeDtypeStruct((N, DC), jnp.float32),
        ],
    )(x, W_in, bb, deg)


def _tc_linscale(h, Wr, dinv):
    def body(h_ref, w_ref, dinv_ref, g_ref):
        acc = jnp.dot(h_ref[0], w_ref[0], precision=_PREC)
        for c in range(1, C):
            acc += jnp.dot(h_ref[c], w_ref[c], precision=_PREC)
        g_ref[0] = dinv_ref[...] * acc

    return pl.pallas_call(
        body,
        grid=(NR, C),
        in_specs=[
            pl.BlockSpec((C, R, DC), lambda i, co: (0, i, 0)),
            pl.BlockSpec((C, DC, DC), lambda i, co: (0, 0, co)),
            pl.BlockSpec((R, DC), lambda i, co: (i, 0)),
        ],
        out_specs=pl.BlockSpec((1, R, DC), lambda i, co: (co, i, 0)),
        out_shape=jax.ShapeDtypeStruct((C, N, DC), jnp.float32),
    )(h, Wr, dinv)


def _tc_combine(part, g, dinv, bb):
    def body(p_ref, g_ref, dinv_ref, b_ref, conv_ref, st_ref):
        i = pl.program_id(1)
        conv = dinv_ref[...] * (p_ref[0, 0] + p_ref[1, 0] + g_ref[0]) \
            + b_ref[0, 0:1, :]
        conv_ref[0] = conv
        ssum = jnp.sum(conv.reshape(R // 8, 8, DC), axis=0)
        ssq = jnp.sum((conv * conv).reshape(R // 8, 8, DC), axis=0)

        @pl.when(i == 0)
        def _():
            st_ref[0, 0] = ssum
            st_ref[0, 1] = ssq

        @pl.when(i != 0)
        def _():
            st_ref[0, 0] += ssum
            st_ref[0, 1] += ssq

    return pl.pallas_call(
        body,
        grid=(C, NR),
        in_specs=[
            pl.BlockSpec((2, 1, R, DC), lambda c, i: (0, c, i, 0)),
            pl.BlockSpec((1, R, DC), lambda c, i: (c, i, 0)),
            pl.BlockSpec((R, DC), lambda c, i: (i, 0)),
            pl.BlockSpec((1, 8, DC), lambda c, i: (c, 0, 0)),
        ],
        out_specs=[
            pl.BlockSpec((1, R, DC), lambda c, i: (c, i, 0)),
            pl.BlockSpec((1, 2, 8, DC), lambda c, i: (c, 0, 0, 0)),
        ],
        out_shape=[
            jax.ShapeDtypeStruct((C, N, DC), jnp.float32),
            jax.ShapeDtypeStruct((C, 2, 8, DC), jnp.float32),
        ],
    )(part, g, dinv, bb)


def _tc_bn_relu_res(conv, stats, gam, bet, hprev):
    def body(conv_ref, st_ref, g_ref, b_ref, hp_ref, h_ref):
        mu = jnp.sum(st_ref[0, 0], axis=0, keepdims=True) * (1.0 / N)
        ex2 = jnp.sum(st_ref[0, 1], axis=0, keepdims=True) * (1.0 / N)
        var = ex2 - mu * mu
        inv = lax.rsqrt(var + 1e-5)
        y = (conv_ref[0] - mu) * inv * g_ref[0, 0:1, :] + b_ref[0, 0:1, :]
        h_ref[0] = jnp.maximum(y, 0.0) + hp_ref[0]

    return pl.pallas_call(
        body,
        grid=(C, NR),
        in_specs=[
            pl.BlockSpec((1, R, DC), lambda c, i: (c, i, 0)),
            pl.BlockSpec((1, 2, 8, DC), lambda c, i: (c, 0, 0, 0)),
            pl.BlockSpec((1, 8, DC), lambda c, i: (c, 0, 0)),
            pl.BlockSpec((1, 8, DC), lambda c, i: (c, 0, 0)),
            pl.BlockSpec((1, R, DC), lambda c, i: (c, i, 0)),
        ],
        out_specs=pl.BlockSpec((1, R, DC), lambda c, i: (c, i, 0)),
        out_shape=jax.ShapeDtypeStruct((C, N, DC), jnp.float32),
    )(conv, stats, gam, bet, hprev)


def _tc_out(h, Wo, bo):
    def body(h_ref, w_ref, b_ref, o_ref):
        acc = jnp.dot(h_ref[0], w_ref[0], precision=_PREC)
        for c in range(1, C):
            acc += jnp.dot(h_ref[c], w_ref[c], precision=_PREC)
        o_ref[...] = acc + b_ref[0:1, :]

    return pl.pallas_call(
        body,
        grid=(NR,),
        in_specs=[
            pl.BlockSpec((C, R, DC), lambda i: (0, i, 0)),
            pl.BlockSpec((C, DC, 128), lambda i: (0, 0, 0)),
            pl.BlockSpec((8, 128), lambda i: (0, 0)),
        ],
        out_specs=pl.BlockSpec((R, 128), lambda i: (i, 0)),
        out_shape=jax.ShapeDtypeStruct((N, 128), jnp.float32),
    )(h, Wo, bo)


def _bc(v):
    return jnp.broadcast_to(v.reshape(C, 1, DC), (C, 8, DC))


def kernel(x, edge_index, W_in, b_in, W1, b1, g1, beta1, W2, b2, g2, beta2,
           W3, b3, g3, beta3, W_out, b_out):
    src = edge_index[0]
    dst = edge_index[1]
    pad = EPAD - E
    src_p = jnp.concatenate([src, jnp.zeros((pad,), jnp.int32)])
    dst_p = jnp.concatenate([dst, jnp.full((pad,), N, jnp.int32)])
    dsts40 = dst_p.reshape(TILES, KB, 128)
    dsts = dst_p.reshape(EPAD // 128, 128)
    srcs = src_p.reshape(EPAD // 128, 128)
    ones128 = jnp.ones((128, DC), jnp.float32)
    zeros128 = jnp.zeros((128, DC), jnp.float32)

    deg = _sc_degree(dsts40, ones128, zeros128)
    h, dinv = _tc_in_proj(x, W_in, _bc(b_in), deg)

    for (W, b, gm, be) in ((W1, b1, g1, beta1), (W2, b2, g2, beta2),
                           (W3, b3, g3, beta3)):
        g = _tc_linscale(h, W.reshape(C, DC, D_H), dinv)
        part = _sc_scatter(srcs, dsts, g.reshape(C * N, DC))
        conv, stats = _tc_combine(part, g, dinv, _bc(b))
        h = _tc_bn_relu_res(conv, stats, _bc(gm), _bc(be), h)

    Wo = jnp.pad(W_out, ((0, 0), (0, 128 - W_out.shape[1])))
    bo = jnp.broadcast_to(jnp.pad(b_out, (0, 128 - b_out.shape[0]))[None, :],
                          (8, 128))
    out128 = _tc_out(h, Wo.reshape(C, DC, 128), bo)
    return out128[:, :b_out.shape[0]]
